# Initial kernel scaffold; baseline (speedup 1.0000x reference)
import jax
import jax.numpy as jnp
from jax.experimental import pallas as pl


def kernel(z_i, z_j, c_i, c_j):
    raise NotImplementedError("write your pallas kernel here")


# speedup vs baseline: 42.7080x; 42.7080x over previous
import jax
import jax.numpy as jnp
from jax.experimental import pallas as pl

_TOPK1 = 33.0
_INV_T = 2.0
_THRESH = 0.5
_B = 4096
_D = 128
_BLK = 256
_NBLK = _B // _BLK
_ITERS = 32


def _body(zi_blk, zj_blk, ci_blk, zi_all, zj_all, ci_all, out_ref):
    step = pl.program_id(0)
    a = zi_blk[...]
    b = zj_blk[...]
    ac = ci_blk[...]
    Zi = zi_all[...]
    Zj = zj_all[...]
    Ci = ci_all[...]

    f32 = jnp.float32
    dot = lambda p, q: jax.lax.dot_general(
        p, q, (((1,), (1,)), ((), ())), preferred_element_type=f32)

    ones_row = jnp.ones((1, _D), dtype=f32)
    inv_a = jax.lax.rsqrt(jnp.maximum(jnp.sum(a * a, axis=1, keepdims=True), 1e-12))
    inv_b = jax.lax.rsqrt(jnp.maximum(jnp.sum(b * b, axis=1, keepdims=True), 1e-12))
    inv_zi = jax.lax.rsqrt(jnp.maximum(dot(ones_row, Zi * Zi), 1e-12))
    inv_zj = jax.lax.rsqrt(jnp.maximum(dot(ones_row, Zj * Zj), 1e-12))

    sim_x = dot(a, Zi) * inv_a * inv_zi
    sim_adv = dot(b, Zj) * inv_b * inv_zj
    sim_ci = dot(a, Zj) * inv_a * inv_zj
    sim_cj = dot(b, Zi) * inv_b * inv_zi

    lo0 = jnp.full((_BLK, 1), -1.01, dtype=f32)
    hi0 = jnp.full((_BLK, 1), 1.01, dtype=f32)

    def bisect(_, carry):
        lo, hi = carry
        mid = (lo + hi) * 0.5
        cnt = jnp.sum(jnp.where(sim_x >= mid, 1.0, 0.0), axis=1, keepdims=True)
        take = cnt >= _TOPK1
        return jnp.where(take, mid, lo), jnp.where(take, hi, mid)

    lo, hi = jax.lax.fori_loop(0, _ITERS, bisect, (lo0, hi0))

    rows = step * _BLK + jax.lax.broadcasted_iota(jnp.int32, (_BLK, 1), 0)
    cols = jax.lax.broadcasted_iota(jnp.int32, (_BLK, _B), 1)
    isdiag = cols == rows
    sel = jnp.where(sim_x >= lo, 1.0, 0.0)
    sel_nd = jnp.where(isdiag, 0.0, sel)

    mm = dot(ac, Ci)
    m = jnp.where(isdiag, 1.0, jnp.where(mm > _THRESH, 1.0, 0.0))

    rsum = lambda v: jnp.sum(v, axis=1, keepdims=True)
    S1 = rsum(sel_nd * jnp.exp(_INV_T * sim_x) + sel * jnp.exp(_INV_T * sim_ci))
    S2 = rsum(sel_nd * jnp.exp(_INV_T * sim_adv) + sel * jnp.exp(_INV_T * sim_cj))
    A1 = _INV_T * rsum(m * (sel_nd * sim_x + sel * sim_ci))
    A2 = _INV_T * rsum(m * (sel_nd * sim_adv + sel * sim_cj))
    denom = 2.0 * rsum(sel_nd * m) + 1.0

    row_loss = (A1 + A2) / denom - jnp.log(S1) - jnp.log(S2)
    s = jnp.sum(row_loss)

    @pl.when(step == 0)
    def _():
        out_ref[...] = jnp.zeros((8, 128), dtype=f32)

    sub = jax.lax.broadcasted_iota(jnp.int32, (8, 128), 0)
    lane = jax.lax.broadcasted_iota(jnp.int32, (8, 128), 1)
    onehot = jnp.where((sub == 0) & (lane == 0), 1.0, 0.0)
    out_ref[...] += s * onehot


def kernel(z_i, z_j, c_i, c_j):
    ci_pad = jnp.pad(c_i, ((0, 0), (0, _D - c_i.shape[1])))
    out = pl.pallas_call(
        _body,
        grid=(_NBLK,),
        in_specs=[
            pl.BlockSpec((_BLK, _D), lambda i: (i, 0)),
            pl.BlockSpec((_BLK, _D), lambda i: (i, 0)),
            pl.BlockSpec((_BLK, _D), lambda i: (i, 0)),
            pl.BlockSpec((_B, _D), lambda i: (0, 0)),
            pl.BlockSpec((_B, _D), lambda i: (0, 0)),
            pl.BlockSpec((_B, _D), lambda i: (0, 0)),
        ],
        out_specs=pl.BlockSpec((8, 128), lambda i: (0, 0)),
        out_shape=jax.ShapeDtypeStruct((8, 128), jnp.float32),
    )(z_i, z_j, ci_pad, z_i, z_j, ci_pad)
    return -out[0, 0] / _B


# speedup vs baseline: 63.8513x; 1.4951x over previous
import jax
import jax.numpy as jnp
from jax.experimental import pallas as pl
from jax.experimental.pallas import tpu as pltpu

_TOPK1 = 33.0
_INV_T = 2.0
_THRESH = 0.5
_B = 4096
_D = 128
_BLK = 256
_NBLK = _B // _BLK
_ITERS = 18


def _body(zi_all, zj_all, ci_blk, ci_all, out_ref, zih, zjh):
    step = pl.program_id(0)
    f32 = jnp.float32
    dot = lambda p, q: jax.lax.dot_general(
        p, q, (((1,), (1,)), ((), ())), preferred_element_type=f32)

    @pl.when(step == 0)
    def _():
        Zi = zi_all[...]
        Zj = zj_all[...]
        inv_i = jax.lax.rsqrt(jnp.maximum(jnp.sum(Zi * Zi, 1, keepdims=True), 1e-12))
        inv_j = jax.lax.rsqrt(jnp.maximum(jnp.sum(Zj * Zj, 1, keepdims=True), 1e-12))
        zih[...] = Zi * inv_i
        zjh[...] = Zj * inv_j

    Zih = zih[...]
    Zjh = zjh[...]
    a = zih[pl.ds(step * _BLK, _BLK), :]
    b = zjh[pl.ds(step * _BLK, _BLK), :]
    ac = ci_blk[...]
    Ci = ci_all[...]

    sx = dot(a, Zih)
    sa = dot(b, Zjh)
    sci = dot(a, Zjh)
    scj = dot(b, Zih)

    ones_c = jnp.ones((1, _B), dtype=f32)

    lo0 = jnp.full((_BLK, 1), -1.01, dtype=f32)
    hi0 = jnp.full((_BLK, 1), 1.01, dtype=f32)

    def bisect(_, carry):
        lo, hi = carry
        mid = (lo + hi) * 0.5
        cnt = dot(jnp.where(sx >= mid, 1.0, 0.0), ones_c)
        take = cnt >= _TOPK1
        return jnp.where(take, mid, lo), jnp.where(take, hi, mid)

    lo, hi = jax.lax.fori_loop(0, _ITERS, bisect, (lo0, hi0))
    sel = jnp.where(sx >= lo, 1.0, 0.0)

    rows = step * _BLK + jax.lax.broadcasted_iota(jnp.int32, (_BLK, _B), 0)
    cols = jax.lax.broadcasted_iota(jnp.int32, (_BLK, _B), 1)
    mm = dot(ac, Ci)
    m = jnp.where(cols == rows, 1.0, jnp.where(mm > _THRESH, 1.0, 0.0))
    ms = m * sel

    dsx = jnp.sum(a * a, axis=1, keepdims=True)
    dsa = jnp.sum(b * b, axis=1, keepdims=True)

    ex = jnp.exp(_INV_T * sx)
    eci = jnp.exp(_INV_T * sci)
    ea = jnp.exp(_INV_T * sa)
    ecj = jnp.exp(_INV_T * scj)

    S1 = dot(sel * (ex + eci), ones_c) - jnp.exp(_INV_T * dsx)
    S2 = dot(sel * (ea + ecj), ones_c) - jnp.exp(_INV_T * dsa)
    A1 = _INV_T * (dot(ms * (sx + sci), ones_c) - dsx)
    A2 = _INV_T * (dot(ms * (sa + scj), ones_c) - dsa)
    denom = 2.0 * dot(ms, ones_c) - 1.0

    row_loss = (A1 + A2) / denom - jnp.log(S1) - jnp.log(S2)
    s = jnp.sum(row_loss)

    @pl.when(step == 0)
    def _():
        out_ref[...] = jnp.zeros((8, 128), dtype=f32)

    sub = jax.lax.broadcasted_iota(jnp.int32, (8, 128), 0)
    lane = jax.lax.broadcasted_iota(jnp.int32, (8, 128), 1)
    onehot = jnp.where((sub == 0) & (lane == 0), 1.0, 0.0)
    out_ref[...] += s * onehot


def kernel(z_i, z_j, c_i, c_j):
    ci_pad = jnp.pad(c_i, ((0, 0), (0, _D - c_i.shape[1])))
    out = pl.pallas_call(
        _body,
        grid=(_NBLK,),
        in_specs=[
            pl.BlockSpec((_B, _D), lambda i: (0, 0)),
            pl.BlockSpec((_B, _D), lambda i: (0, 0)),
            pl.BlockSpec((_BLK, _D), lambda i: (i, 0)),
            pl.BlockSpec((_B, _D), lambda i: (0, 0)),
        ],
        out_specs=pl.BlockSpec((8, 128), lambda i: (0, 0)),
        out_shape=jax.ShapeDtypeStruct((8, 128), jnp.float32),
        scratch_shapes=[
            pltpu.VMEM((_B, _D), jnp.float32),
            pltpu.VMEM((_B, _D), jnp.float32),
        ],
    )(z_i, z_j, ci_pad, ci_pad)
    return -out[0, 0] / _B


# speedup vs baseline: 88.0771x; 1.3794x over previous
import jax
import jax.numpy as jnp
from jax.experimental import pallas as pl
from jax.experimental.pallas import tpu as pltpu

_TOPK1 = 33.0
_INV_T = 2.0
_THRESH = 0.5
_B = 4096
_D = 128
_BLK = 256
_NBLK = _B // _BLK
_ITERS = 14


def _body(zi_all, zj_all, ci_blk, ci_all, out_ref, zih, zjh):
    step = pl.program_id(0)
    f32 = jnp.float32
    dot = lambda p, q: jax.lax.dot_general(
        p, q, (((1,), (1,)), ((), ())), preferred_element_type=f32)

    @pl.when(step == 0)
    def _():
        Zi = zi_all[...]
        Zj = zj_all[...]
        inv_i = jax.lax.rsqrt(jnp.maximum(jnp.sum(Zi * Zi, 1, keepdims=True), 1e-12))
        inv_j = jax.lax.rsqrt(jnp.maximum(jnp.sum(Zj * Zj, 1, keepdims=True), 1e-12))
        zih[...] = Zi * inv_i
        zjh[...] = Zj * inv_j

    Zih = zih[...]
    Zjh = zjh[...]
    a = zih[pl.ds(step * _BLK, _BLK), :]
    b = zjh[pl.ds(step * _BLK, _BLK), :]
    ac = ci_blk[...]
    Ci = ci_all[...]

    sx = dot(a, Zih)
    sa = dot(b, Zjh)
    sci = dot(a, Zjh)
    scj = dot(b, Zih)

    ones_c = jnp.ones((1, _B), dtype=f32)

    lo0 = jnp.full((_BLK, 1), -1.01, dtype=f32)
    hi0 = jnp.full((_BLK, 1), 1.01, dtype=f32)

    lo, hi = lo0, hi0
    for _ in range(_ITERS):
        mid = (lo + hi) * 0.5
        cnt = dot(jnp.where(sx >= mid, 1.0, 0.0), ones_c)
        take = cnt >= _TOPK1
        lo, hi = jnp.where(take, mid, lo), jnp.where(take, hi, mid)
    sel = jnp.where(sx >= lo, 1.0, 0.0)

    rows = step * _BLK + jax.lax.broadcasted_iota(jnp.int32, (_BLK, _B), 0)
    cols = jax.lax.broadcasted_iota(jnp.int32, (_BLK, _B), 1)
    mm = dot(ac, Ci)
    m = jnp.where(cols == rows, 1.0, jnp.where(mm > _THRESH, 1.0, 0.0))
    ms = m * sel

    dsx = jnp.sum(a * a, axis=1, keepdims=True)
    dsa = jnp.sum(b * b, axis=1, keepdims=True)

    ex = jnp.exp(_INV_T * sx)
    eci = jnp.exp(_INV_T * sci)
    ea = jnp.exp(_INV_T * sa)
    ecj = jnp.exp(_INV_T * scj)

    S1 = dot(sel * (ex + eci), ones_c) - jnp.exp(_INV_T * dsx)
    S2 = dot(sel * (ea + ecj), ones_c) - jnp.exp(_INV_T * dsa)
    A1 = _INV_T * (dot(ms * (sx + sci), ones_c) - dsx)
    A2 = _INV_T * (dot(ms * (sa + scj), ones_c) - dsa)
    denom = 2.0 * dot(ms, ones_c) - 1.0

    row_loss = (A1 + A2) / denom - jnp.log(S1) - jnp.log(S2)
    s = jnp.sum(row_loss)

    @pl.when(step == 0)
    def _():
        out_ref[...] = jnp.zeros((8, 128), dtype=f32)

    sub = jax.lax.broadcasted_iota(jnp.int32, (8, 128), 0)
    lane = jax.lax.broadcasted_iota(jnp.int32, (8, 128), 1)
    onehot = jnp.where((sub == 0) & (lane == 0), 1.0, 0.0)
    out_ref[...] += s * onehot


def kernel(z_i, z_j, c_i, c_j):
    ci_pad = jnp.pad(c_i, ((0, 0), (0, _D - c_i.shape[1])))
    out = pl.pallas_call(
        _body,
        grid=(_NBLK,),
        in_specs=[
            pl.BlockSpec((_B, _D), lambda i: (0, 0)),
            pl.BlockSpec((_B, _D), lambda i: (0, 0)),
            pl.BlockSpec((_BLK, _D), lambda i: (i, 0)),
            pl.BlockSpec((_B, _D), lambda i: (0, 0)),
        ],
        out_specs=pl.BlockSpec((8, 128), lambda i: (0, 0)),
        out_shape=jax.ShapeDtypeStruct((8, 128), jnp.float32),
        scratch_shapes=[
            pltpu.VMEM((_B, _D), jnp.float32),
            pltpu.VMEM((_B, _D), jnp.float32),
        ],
    )(z_i, z_j, ci_pad, ci_pad)
    return -out[0, 0] / _B
